# Initial kernel scaffold; baseline (speedup 1.0000x reference)
import jax
import jax.numpy as jnp
from jax.experimental import pallas as pl


def kernel(x, W_r, b_r, W_e, b_e, W_q, b_q, W_a, b_a):
    raise NotImplementedError("write your pallas kernel here")


# speedup vs baseline: 1.5382x; 1.5382x over previous
import functools

import jax
import jax.numpy as jnp
from jax.experimental import pallas as pl

N_TOK = 2048
D_IN = 1024
HID = 1024
D_OUT = 1024
NUM_CHUNKS = 8
TOP_K = 2
THRESHOLD = 0.2


def _router_kernel(x_ref, wr_ref, br_ref, gated_ref, cact_ref, mact_ref):
    x = x_ref[...]
    logits = jnp.dot(x, wr_ref[...], preferred_element_type=jnp.float32,
                     precision=jax.lax.Precision.HIGHEST) + br_ref[...]
    m = jnp.max(logits, axis=-1, keepdims=True)
    e = jnp.exp(logits - m)
    gates = e / jnp.sum(e, axis=-1, keepdims=True)
    c_iota = jax.lax.broadcasted_iota(jnp.int32, gates.shape, 1)
    m1 = jnp.max(gates, axis=-1, keepdims=True)
    i1 = jnp.min(jnp.where(gates == m1, c_iota, NUM_CHUNKS), axis=-1,
                 keepdims=True)
    mask1 = c_iota == i1
    g2 = jnp.where(mask1, -jnp.inf, gates)
    m2 = jnp.max(g2, axis=-1, keepdims=True)
    i2 = jnp.min(jnp.where(g2 == m2, c_iota, NUM_CHUNKS), axis=-1,
                 keepdims=True)
    mask = mask1 | (c_iota == i2)
    gated = jnp.where(mask, gates, 0.0)
    gated_ref[...] = gated
    cact = jnp.sum(gated, axis=0, keepdims=True) * (1.0 / N_TOK)
    cact_ref[...] = cact
    mact_ref[...] = jnp.sum(cact, axis=1, keepdims=True) * (1.0 / NUM_CHUNKS)


def _expert_kernel(x_ref, we_ref, be_ref, gated_ref, x2_ref):
    c = pl.program_id(0)
    xb = x_ref[...].astype(jnp.bfloat16)
    wb = we_ref[0].astype(jnp.bfloat16)
    y = jnp.dot(xb, wb, preferred_element_type=jnp.float32)
    gated = gated_ref[...]
    c_iota = jax.lax.broadcasted_iota(jnp.int32, gated.shape, 1)
    g = jnp.sum(jnp.where(c_iota == c, gated, 0.0), axis=1, keepdims=True)
    term = g * (y + be_ref[0])

    @pl.when(c == 0)
    def _():
        x2_ref[...] = term

    @pl.when(c > 0)
    def _():
        x2_ref[...] += term


def _quant_kernel(x2_ref, wq_ref, bq_ref, mact_ref, x3_ref, act_ref):
    m = mact_ref[0, 0]
    w = wq_ref[...]
    scale = jnp.max(jnp.abs(w)) * (1.0 / 127.0)
    w_fq = jnp.round(w / scale) * scale
    w_blend = (m * w + (1.0 - m) * w_fq).astype(jnp.bfloat16)
    x3 = jnp.dot(x2_ref[...].astype(jnp.bfloat16), w_blend,
                 preferred_element_type=jnp.float32) + bq_ref[...]
    x3_ref[...] = x3
    act_ref[...] = jnp.sum(jnp.abs(x3), axis=(0, 1), keepdims=True) * (
        1.0 / (N_TOK * HID))


def _final_kernel(x3_ref, wa_ref, ba_ref, out_ref):
    out_ref[...] = jnp.dot(x3_ref[...].astype(jnp.bfloat16),
                           wa_ref[...].astype(jnp.bfloat16),
                           preferred_element_type=jnp.float32) + ba_ref[...]


@jax.jit
def kernel(x, W_r, b_r, W_e, b_e, W_q, b_q, W_a, b_a):
    gated, cact, mact = pl.pallas_call(
        _router_kernel,
        out_shape=(
            jax.ShapeDtypeStruct((N_TOK, NUM_CHUNKS), jnp.float32),
            jax.ShapeDtypeStruct((1, NUM_CHUNKS), jnp.float32),
            jax.ShapeDtypeStruct((1, 1), jnp.float32),
        ),
    )(x, W_r, b_r.reshape(1, NUM_CHUNKS))

    x2 = pl.pallas_call(
        _expert_kernel,
        grid=(NUM_CHUNKS,),
        in_specs=[
            pl.BlockSpec((N_TOK, D_IN), lambda c: (0, 0)),
            pl.BlockSpec((1, D_IN, HID), lambda c: (c, 0, 0)),
            pl.BlockSpec((1, 1, HID), lambda c: (c, 0, 0)),
            pl.BlockSpec((N_TOK, NUM_CHUNKS), lambda c: (0, 0)),
        ],
        out_specs=pl.BlockSpec((N_TOK, HID), lambda c: (0, 0)),
        out_shape=jax.ShapeDtypeStruct((N_TOK, HID), jnp.float32),
    )(x, W_e, b_e.reshape(NUM_CHUNKS, 1, HID), gated)

    x3, act = pl.pallas_call(
        _quant_kernel,
        out_shape=(
            jax.ShapeDtypeStruct((N_TOK, HID), jnp.float32),
            jax.ShapeDtypeStruct((1, 1), jnp.float32),
        ),
    )(x2, W_q, b_q.reshape(1, HID), mact)

    def _full(x3_):
        return pl.pallas_call(
            _final_kernel,
            out_shape=jax.ShapeDtypeStruct((N_TOK, D_OUT), jnp.float32),
        )(x3_, W_a, b_a.reshape(1, D_OUT))

    act_s = act[0, 0]
    out = jax.lax.cond(act_s > THRESHOLD, _full,
                       lambda x3_: jnp.zeros((N_TOK, D_OUT), jnp.float32),
                       x3)
    return (out, cact.reshape(NUM_CHUNKS), mact[0, 0], act_s)


# speedup vs baseline: 1.7452x; 1.1345x over previous
import jax
import jax.numpy as jnp
from jax.experimental import pallas as pl
from jax.experimental.pallas import tpu as pltpu

N_TOK = 2048
D_IN = 1024
HID = 1024
D_OUT = 1024
NUM_CHUNKS = 8
TOP_K = 2
THRESHOLD = 0.2


def _fused_kernel(x_ref, wr_ref, br_ref, we_ref, be_ref, wq_ref, bq_ref,
                  x3_ref, cact_ref, mact_ref, act_ref,
                  xb_s, x2_s, gated_s):
    c = pl.program_id(0)

    @pl.when(c == 0)
    def _router():
        x = x_ref[...]
        xb_s[...] = x.astype(jnp.bfloat16)
        logits = jnp.dot(x, wr_ref[...], preferred_element_type=jnp.float32,
                         precision=jax.lax.Precision.HIGHEST) + br_ref[...]
        m = jnp.max(logits, axis=-1, keepdims=True)
        e = jnp.exp(logits - m)
        gates = e / jnp.sum(e, axis=-1, keepdims=True)
        c_iota = jax.lax.broadcasted_iota(jnp.int32, gates.shape, 1)
        m1 = jnp.max(gates, axis=-1, keepdims=True)
        i1 = jnp.min(jnp.where(gates == m1, c_iota, NUM_CHUNKS), axis=-1,
                     keepdims=True)
        mask1 = c_iota == i1
        g2 = jnp.where(mask1, -jnp.inf, gates)
        m2 = jnp.max(g2, axis=-1, keepdims=True)
        i2 = jnp.min(jnp.where(g2 == m2, c_iota, NUM_CHUNKS), axis=-1,
                     keepdims=True)
        mask = mask1 | (c_iota == i2)
        gated = jnp.where(mask, gates, 0.0)
        gated_s[...] = gated
        cact = jnp.sum(gated, axis=0, keepdims=True) * (1.0 / N_TOK)
        cact_ref[...] = cact
        mact_ref[...] = jnp.sum(cact, axis=1, keepdims=True) * (
            1.0 / NUM_CHUNKS)

    @pl.when(c < NUM_CHUNKS)
    def _expert():
        y = jnp.dot(xb_s[...], we_ref[0].astype(jnp.bfloat16),
                    preferred_element_type=jnp.float32)
        gated = gated_s[...]
        c_iota = jax.lax.broadcasted_iota(jnp.int32, gated.shape, 1)
        g = jnp.sum(jnp.where(c_iota == c, gated, 0.0), axis=1, keepdims=True)
        term = g * (y + be_ref[0])

        @pl.when(c == 0)
        def _():
            x2_s[...] = term

        @pl.when(c > 0)
        def _():
            x2_s[...] += term

    @pl.when(c == NUM_CHUNKS)
    def _quant():
        m = mact_ref[0, 0]
        w = wq_ref[...]
        scale = jnp.max(jnp.abs(w)) * (1.0 / 127.0)
        w_fq = jnp.round(w / scale) * scale
        w_blend = (m * w + (1.0 - m) * w_fq).astype(jnp.bfloat16)
        x3 = jnp.dot(x2_s[...].astype(jnp.bfloat16), w_blend,
                     preferred_element_type=jnp.float32) + bq_ref[...]
        x3_ref[...] = x3
        act_ref[...] = jnp.sum(jnp.abs(x3), axis=(0, 1), keepdims=True) * (
            1.0 / (N_TOK * HID))


def _final_kernel(x3_ref, wa_ref, ba_ref, out_ref):
    out_ref[...] = jnp.dot(x3_ref[...].astype(jnp.bfloat16),
                           wa_ref[...].astype(jnp.bfloat16),
                           preferred_element_type=jnp.float32) + ba_ref[...]


@jax.jit
def kernel(x, W_r, b_r, W_e, b_e, W_q, b_q, W_a, b_a):
    x3, cact, mact, act = pl.pallas_call(
        _fused_kernel,
        grid=(NUM_CHUNKS + 1,),
        in_specs=[
            pl.BlockSpec((N_TOK, D_IN), lambda c: (0, 0)),
            pl.BlockSpec((D_IN, NUM_CHUNKS), lambda c: (0, 0)),
            pl.BlockSpec((1, NUM_CHUNKS), lambda c: (0, 0)),
            pl.BlockSpec((1, D_IN, HID),
                         lambda c: (jnp.minimum(c, NUM_CHUNKS - 1), 0, 0)),
            pl.BlockSpec((1, 1, HID),
                         lambda c: (jnp.minimum(c, NUM_CHUNKS - 1), 0, 0)),
            pl.BlockSpec((HID, HID), lambda c: (0, 0)),
            pl.BlockSpec((1, HID), lambda c: (0, 0)),
        ],
        out_specs=(
            pl.BlockSpec((N_TOK, HID), lambda c: (0, 0)),
            pl.BlockSpec((1, NUM_CHUNKS), lambda c: (0, 0)),
            pl.BlockSpec((1, 1), lambda c: (0, 0)),
            pl.BlockSpec((1, 1), lambda c: (0, 0)),
        ),
        out_shape=(
            jax.ShapeDtypeStruct((N_TOK, HID), jnp.float32),
            jax.ShapeDtypeStruct((1, NUM_CHUNKS), jnp.float32),
            jax.ShapeDtypeStruct((1, 1), jnp.float32),
            jax.ShapeDtypeStruct((1, 1), jnp.float32),
        ),
        scratch_shapes=[
            pltpu.VMEM((N_TOK, D_IN), jnp.bfloat16),
            pltpu.VMEM((N_TOK, HID), jnp.float32),
            pltpu.VMEM((N_TOK, NUM_CHUNKS), jnp.float32),
        ],
    )(x, W_r, b_r.reshape(1, NUM_CHUNKS), W_e,
      b_e.reshape(NUM_CHUNKS, 1, HID), W_q, b_q.reshape(1, HID))

    def _full(x3_):
        return pl.pallas_call(
            _final_kernel,
            out_shape=jax.ShapeDtypeStruct((N_TOK, D_OUT), jnp.float32),
        )(x3_, W_a, b_a.reshape(1, D_OUT))

    act_s = act[0, 0]
    out = jax.lax.cond(act_s > THRESHOLD, _full,
                       lambda x3_: jnp.zeros((N_TOK, D_OUT), jnp.float32),
                       x3)
    return (out, cact.reshape(NUM_CHUNKS), mact[0, 0], act_s)


# speedup vs baseline: 1.7895x; 1.0254x over previous
import jax
import jax.numpy as jnp
from jax.experimental import pallas as pl
from jax.experimental.pallas import tpu as pltpu

N_TOK = 2048
D_IN = 1024
HID = 1024
D_OUT = 1024
NUM_CHUNKS = 8
TOP_K = 2
THRESHOLD = 0.2


def _fused_kernel(x_ref, wr_ref, br_ref, we_ref, be_ref, wq_ref, bq_ref,
                  x3_ref, cact_ref, mact_ref, act_ref,
                  xb_s, x2_s, gated_s):
    c = pl.program_id(0)

    @pl.when(c == 0)
    def _router():
        x = x_ref[...]
        xb_s[...] = x.astype(jnp.bfloat16)
        logits = jnp.dot(x, wr_ref[...], preferred_element_type=jnp.float32,
                         precision=jax.lax.Precision.HIGHEST) + br_ref[...]
        m = jnp.max(logits, axis=-1, keepdims=True)
        e = jnp.exp(logits - m)
        gates = e / jnp.sum(e, axis=-1, keepdims=True)
        c_iota = jax.lax.broadcasted_iota(jnp.int32, gates.shape, 1)
        m1 = jnp.max(gates, axis=-1, keepdims=True)
        i1 = jnp.min(jnp.where(gates == m1, c_iota, NUM_CHUNKS), axis=-1,
                     keepdims=True)
        mask1 = c_iota == i1
        g2 = jnp.where(mask1, -jnp.inf, gates)
        m2 = jnp.max(g2, axis=-1, keepdims=True)
        i2 = jnp.min(jnp.where(g2 == m2, c_iota, NUM_CHUNKS), axis=-1,
                     keepdims=True)
        mask = mask1 | (c_iota == i2)
        gated = jnp.where(mask, gates, 0.0)
        gated_s[...] = gated
        cact = jnp.sum(gated, axis=0, keepdims=True) * (1.0 / N_TOK)
        cact_ref[...] = cact
        mact_ref[...] = jnp.sum(cact, axis=1, keepdims=True) * (
            1.0 / NUM_CHUNKS)

    @pl.when(c < NUM_CHUNKS)
    def _expert():
        gated = gated_s[...]
        c_iota = jax.lax.broadcasted_iota(jnp.int32, gated.shape, 1)
        g = jnp.sum(jnp.where(c_iota == c, gated, 0.0), axis=1, keepdims=True)
        xg = xb_s[...] * g.astype(jnp.bfloat16)
        y = jnp.dot(xg, we_ref[0].astype(jnp.bfloat16),
                    preferred_element_type=jnp.float32)

        @pl.when(c == 0)
        def _():
            x2_s[...] = y

        @pl.when(c > 0)
        def _():
            x2_s[...] += y

    @pl.when(c == NUM_CHUNKS)
    def _quant():
        m = mact_ref[0, 0]
        w = wq_ref[...]
        scale = jnp.max(jnp.abs(w)) * (1.0 / 127.0)
        w_fq = jnp.round(w / scale) * scale
        w_blend = (m * w + (1.0 - m) * w_fq).astype(jnp.bfloat16)
        x2 = x2_s[...] + jnp.dot(gated_s[...].astype(jnp.bfloat16),
                                 be_ref[...].astype(jnp.bfloat16),
                                 preferred_element_type=jnp.float32)
        x3 = jnp.dot(x2.astype(jnp.bfloat16), w_blend,
                     preferred_element_type=jnp.float32) + bq_ref[...]
        x3_ref[...] = x3
        act_ref[...] = jnp.sum(jnp.abs(x3), axis=(0, 1), keepdims=True) * (
            1.0 / (N_TOK * HID))


def _final_kernel(x3_ref, wa_ref, ba_ref, out_ref):
    out_ref[...] = jnp.dot(x3_ref[...].astype(jnp.bfloat16),
                           wa_ref[...].astype(jnp.bfloat16),
                           preferred_element_type=jnp.float32) + ba_ref[...]


@jax.jit
def kernel(x, W_r, b_r, W_e, b_e, W_q, b_q, W_a, b_a):
    x3, cact, mact, act = pl.pallas_call(
        _fused_kernel,
        grid=(NUM_CHUNKS + 1,),
        in_specs=[
            pl.BlockSpec((N_TOK, D_IN), lambda c: (0, 0)),
            pl.BlockSpec((D_IN, NUM_CHUNKS), lambda c: (0, 0)),
            pl.BlockSpec((1, NUM_CHUNKS), lambda c: (0, 0)),
            pl.BlockSpec((1, D_IN, HID),
                         lambda c: (jnp.minimum(c, NUM_CHUNKS - 1), 0, 0)),
            pl.BlockSpec((NUM_CHUNKS, HID), lambda c: (0, 0)),
            pl.BlockSpec((HID, HID), lambda c: (0, 0)),
            pl.BlockSpec((1, HID), lambda c: (0, 0)),
        ],
        out_specs=(
            pl.BlockSpec((N_TOK, HID), lambda c: (0, 0)),
            pl.BlockSpec((1, NUM_CHUNKS), lambda c: (0, 0)),
            pl.BlockSpec((1, 1), lambda c: (0, 0)),
            pl.BlockSpec((1, 1), lambda c: (0, 0)),
        ),
        out_shape=(
            jax.ShapeDtypeStruct((N_TOK, HID), jnp.float32),
            jax.ShapeDtypeStruct((1, NUM_CHUNKS), jnp.float32),
            jax.ShapeDtypeStruct((1, 1), jnp.float32),
            jax.ShapeDtypeStruct((1, 1), jnp.float32),
        ),
        scratch_shapes=[
            pltpu.VMEM((N_TOK, D_IN), jnp.bfloat16),
            pltpu.VMEM((N_TOK, HID), jnp.float32),
            pltpu.VMEM((N_TOK, NUM_CHUNKS), jnp.float32),
        ],
    )(x, W_r, b_r.reshape(1, NUM_CHUNKS), W_e, b_e, W_q, b_q.reshape(1, HID))

    def _full(x3_):
        return pl.pallas_call(
            _final_kernel,
            out_shape=jax.ShapeDtypeStruct((N_TOK, D_OUT), jnp.float32),
        )(x3_, W_a, b_a.reshape(1, D_OUT))

    act_s = act[0, 0]
    out = jax.lax.cond(act_s > THRESHOLD, _full,
                       lambda x3_: jnp.zeros((N_TOK, D_OUT), jnp.float32),
                       x3)
    return (out, cact.reshape(NUM_CHUNKS), mact[0, 0], act_s)


# speedup vs baseline: 1.9351x; 1.0814x over previous
import jax
import jax.numpy as jnp
from jax.experimental import pallas as pl
from jax.experimental.pallas import tpu as pltpu

N_TOK = 2048
D_IN = 1024
HID = 1024
D_OUT = 1024
NUM_CHUNKS = 8
TOP_K = 2
THRESHOLD = 0.2


def _fused_kernel(x_ref, wr_ref, br_ref, we_ref, be_ref, wq_ref, bq_ref,
                  x3_ref, cact_ref, mact_ref, act_ref,
                  xb_s, x2_s, gated_s):
    c = pl.program_id(0)

    @pl.when(c == 0)
    def _router():
        x = x_ref[...]
        xb_s[...] = x.astype(jnp.bfloat16)
        logits = jnp.dot(x, wr_ref[...],
                         preferred_element_type=jnp.float32) + br_ref[...]
        m = jnp.max(logits, axis=-1, keepdims=True)
        e = jnp.exp(logits - m)
        gates = e / jnp.sum(e, axis=-1, keepdims=True)
        c_iota = jax.lax.broadcasted_iota(jnp.int32, gates.shape, 1)
        m1 = jnp.max(gates, axis=-1, keepdims=True)
        i1 = jnp.min(jnp.where(gates == m1, c_iota, NUM_CHUNKS), axis=-1,
                     keepdims=True)
        mask1 = c_iota == i1
        g2 = jnp.where(mask1, -jnp.inf, gates)
        m2 = jnp.max(g2, axis=-1, keepdims=True)
        i2 = jnp.min(jnp.where(g2 == m2, c_iota, NUM_CHUNKS), axis=-1,
                     keepdims=True)
        mask = mask1 | (c_iota == i2)
        gated = jnp.where(mask, gates, 0.0)
        gated_s[...] = gated
        cact = jnp.sum(gated, axis=0, keepdims=True) * (1.0 / N_TOK)
        cact_ref[...] = cact
        mact_ref[...] = jnp.sum(cact, axis=1, keepdims=True) * (
            1.0 / NUM_CHUNKS)

    @pl.when(c < NUM_CHUNKS)
    def _expert():
        gated = gated_s[...]
        c_iota = jax.lax.broadcasted_iota(jnp.int32, gated.shape, 1)
        g = jnp.sum(jnp.where(c_iota == c, gated, 0.0), axis=1, keepdims=True)
        xg = xb_s[...] * g.astype(jnp.bfloat16)
        y = jnp.dot(xg, we_ref[0].astype(jnp.bfloat16),
                    preferred_element_type=jnp.float32)

        @pl.when(c == 0)
        def _():
            x2_s[...] = y

        @pl.when(c > 0)
        def _():
            x2_s[...] += y

    @pl.when(c == NUM_CHUNKS)
    def _quant():
        m = mact_ref[0, 0]
        w = wq_ref[...]
        scale = jnp.max(jnp.abs(w)) * (1.0 / 127.0)
        w_fq = jnp.round(w / scale) * scale
        w_blend = (m * w + (1.0 - m) * w_fq).astype(jnp.bfloat16)
        x2 = x2_s[...] + jnp.dot(gated_s[...].astype(jnp.bfloat16),
                                 be_ref[...].astype(jnp.bfloat16),
                                 preferred_element_type=jnp.float32)
        x3 = jnp.dot(x2.astype(jnp.bfloat16), w_blend,
                     preferred_element_type=jnp.float32) + bq_ref[...]
        x3_ref[...] = x3
        act_ref[...] = jnp.sum(jnp.abs(x3), axis=(0, 1), keepdims=True) * (
            1.0 / (N_TOK * HID))


def _final_kernel(x3_ref, wa_ref, ba_ref, out_ref):
    out_ref[...] = jnp.dot(x3_ref[...].astype(jnp.bfloat16),
                           wa_ref[...].astype(jnp.bfloat16),
                           preferred_element_type=jnp.float32) + ba_ref[...]


@jax.jit
def kernel(x, W_r, b_r, W_e, b_e, W_q, b_q, W_a, b_a):
    x3, cact, mact, act = pl.pallas_call(
        _fused_kernel,
        grid=(NUM_CHUNKS + 1,),
        in_specs=[
            pl.BlockSpec((N_TOK, D_IN), lambda c: (0, 0)),
            pl.BlockSpec((D_IN, NUM_CHUNKS), lambda c: (0, 0)),
            pl.BlockSpec((1, NUM_CHUNKS), lambda c: (0, 0)),
            pl.BlockSpec((1, D_IN, HID),
                         lambda c: (jnp.minimum(c, NUM_CHUNKS - 1), 0, 0)),
            pl.BlockSpec((NUM_CHUNKS, HID), lambda c: (0, 0)),
            pl.BlockSpec((HID, HID), lambda c: (0, 0)),
            pl.BlockSpec((1, HID), lambda c: (0, 0)),
        ],
        out_specs=(
            pl.BlockSpec((N_TOK, HID), lambda c: (0, 0)),
            pl.BlockSpec((1, NUM_CHUNKS), lambda c: (0, 0)),
            pl.BlockSpec((1, 1), lambda c: (0, 0)),
            pl.BlockSpec((1, 1), lambda c: (0, 0)),
        ),
        out_shape=(
            jax.ShapeDtypeStruct((N_TOK, HID), jnp.float32),
            jax.ShapeDtypeStruct((1, NUM_CHUNKS), jnp.float32),
            jax.ShapeDtypeStruct((1, 1), jnp.float32),
            jax.ShapeDtypeStruct((1, 1), jnp.float32),
        ),
        scratch_shapes=[
            pltpu.VMEM((N_TOK, D_IN), jnp.bfloat16),
            pltpu.VMEM((N_TOK, HID), jnp.float32),
            pltpu.VMEM((N_TOK, NUM_CHUNKS), jnp.float32),
        ],
    )(x, W_r, b_r.reshape(1, NUM_CHUNKS), W_e, b_e, W_q, b_q.reshape(1, HID))

    def _full(x3_):
        return pl.pallas_call(
            _final_kernel,
            out_shape=jax.ShapeDtypeStruct((N_TOK, D_OUT), jnp.float32),
        )(x3_, W_a, b_a.reshape(1, D_OUT))

    act_s = act[0, 0]
    out = jax.lax.cond(act_s > THRESHOLD, _full,
                       lambda x3_: jnp.zeros((N_TOK, D_OUT), jnp.float32),
                       x3)
    return (out, cact.reshape(NUM_CHUNKS), mact[0, 0], act_s)


# speedup vs baseline: 2.0199x; 1.0438x over previous
import jax
import jax.numpy as jnp
from jax.experimental import pallas as pl
from jax.experimental.pallas import tpu as pltpu

N_TOK = 2048
D_IN = 1024
HID = 1024
D_OUT = 1024
NUM_CHUNKS = 8
TOP_K = 2
THRESHOLD = 0.2


def _fused_kernel(x_ref, wr_ref, br_ref, we_ref, be_ref, wq_ref, bq_ref,
                  wa_ref, ba_ref,
                  cact_ref, mact_ref, act_ref, out_ref,
                  xb_s, x2_s, gated_s):
    c = pl.program_id(0)

    @pl.when(c == 0)
    def _router():
        x = x_ref[...]
        xb_s[...] = x.astype(jnp.bfloat16)
        logits = jnp.dot(x, wr_ref[...],
                         preferred_element_type=jnp.float32) + br_ref[...]
        m = jnp.max(logits, axis=-1, keepdims=True)
        e = jnp.exp(logits - m)
        gates = e / jnp.sum(e, axis=-1, keepdims=True)
        c_iota = jax.lax.broadcasted_iota(jnp.int32, gates.shape, 1)
        m1 = jnp.max(gates, axis=-1, keepdims=True)
        i1 = jnp.min(jnp.where(gates == m1, c_iota, NUM_CHUNKS), axis=-1,
                     keepdims=True)
        mask1 = c_iota == i1
        g2 = jnp.where(mask1, -jnp.inf, gates)
        m2 = jnp.max(g2, axis=-1, keepdims=True)
        i2 = jnp.min(jnp.where(g2 == m2, c_iota, NUM_CHUNKS), axis=-1,
                     keepdims=True)
        mask = mask1 | (c_iota == i2)
        gated = jnp.where(mask, gates, 0.0)
        gated_s[...] = gated
        cact = jnp.sum(gated, axis=0, keepdims=True) * (1.0 / N_TOK)
        cact_ref[...] = cact
        mact_ref[...] = jnp.sum(cact, axis=1, keepdims=True) * (
            1.0 / NUM_CHUNKS)

    @pl.when(c < NUM_CHUNKS)
    def _expert():
        gated = gated_s[...]
        c_iota = jax.lax.broadcasted_iota(jnp.int32, gated.shape, 1)
        g = jnp.sum(jnp.where(c_iota == c, gated, 0.0), axis=1, keepdims=True)
        xg = xb_s[...] * g.astype(jnp.bfloat16)
        y = jnp.dot(xg, we_ref[0].astype(jnp.bfloat16),
                    preferred_element_type=jnp.float32)

        @pl.when(c == 0)
        def _():
            x2_s[...] = y

        @pl.when(c > 0)
        def _():
            x2_s[...] += y

    @pl.when(c == NUM_CHUNKS)
    def _quant():
        m = mact_ref[0, 0]
        w = wq_ref[...]
        scale = jnp.max(jnp.abs(w)) * (1.0 / 127.0)
        w_fq = jnp.round(w / scale) * scale
        w_blend = (m * w + (1.0 - m) * w_fq).astype(jnp.bfloat16)
        x2 = x2_s[...] + jnp.dot(gated_s[...].astype(jnp.bfloat16),
                                 be_ref[...].astype(jnp.bfloat16),
                                 preferred_element_type=jnp.float32)
        x3 = jnp.dot(x2.astype(jnp.bfloat16), w_blend,
                     preferred_element_type=jnp.float32) + bq_ref[...]
        x2_s[...] = x3
        act_ref[...] = jnp.sum(jnp.abs(x3), axis=(0, 1), keepdims=True) * (
            1.0 / (N_TOK * HID))

    @pl.when(c == NUM_CHUNKS + 1)
    def _final():
        act = act_ref[0, 0]

        @pl.when(act > THRESHOLD)
        def _():
            out_ref[...] = jnp.dot(x2_s[...].astype(jnp.bfloat16),
                                   wa_ref[...].astype(jnp.bfloat16),
                                   preferred_element_type=jnp.float32
                                   ) + ba_ref[...]

        @pl.when(act <= THRESHOLD)
        def _():
            out_ref[...] = jnp.zeros((N_TOK, D_OUT), jnp.float32)


@jax.jit
def kernel(x, W_r, b_r, W_e, b_e, W_q, b_q, W_a, b_a):
    cact, mact, act, out = pl.pallas_call(
        _fused_kernel,
        grid=(NUM_CHUNKS + 2,),
        in_specs=[
            pl.BlockSpec((N_TOK, D_IN), lambda c: (0, 0)),
            pl.BlockSpec((D_IN, NUM_CHUNKS), lambda c: (0, 0)),
            pl.BlockSpec((1, NUM_CHUNKS), lambda c: (0, 0)),
            pl.BlockSpec((1, D_IN, HID),
                         lambda c: (jnp.minimum(c, NUM_CHUNKS - 1), 0, 0)),
            pl.BlockSpec((NUM_CHUNKS, HID), lambda c: (0, 0)),
            pl.BlockSpec((HID, HID), lambda c: (0, 0)),
            pl.BlockSpec((1, HID), lambda c: (0, 0)),
            pl.BlockSpec((HID, D_OUT), lambda c: (0, 0)),
            pl.BlockSpec((1, D_OUT), lambda c: (0, 0)),
        ],
        out_specs=(
            pl.BlockSpec((1, NUM_CHUNKS), lambda c: (0, 0)),
            pl.BlockSpec((1, 1), lambda c: (0, 0)),
            pl.BlockSpec((1, 1), lambda c: (0, 0)),
            pl.BlockSpec((N_TOK, D_OUT), lambda c: (0, 0)),
        ),
        out_shape=(
            jax.ShapeDtypeStruct((1, NUM_CHUNKS), jnp.float32),
            jax.ShapeDtypeStruct((1, 1), jnp.float32),
            jax.ShapeDtypeStruct((1, 1), jnp.float32),
            jax.ShapeDtypeStruct((N_TOK, D_OUT), jnp.float32),
        ),
        scratch_shapes=[
            pltpu.VMEM((N_TOK, D_IN), jnp.bfloat16),
            pltpu.VMEM((N_TOK, HID), jnp.float32),
            pltpu.VMEM((N_TOK, NUM_CHUNKS), jnp.float32),
        ],
    )(x, W_r, b_r.reshape(1, NUM_CHUNKS), W_e, b_e, W_q, b_q.reshape(1, HID),
      W_a, b_a.reshape(1, D_OUT))

    return (out, cact.reshape(NUM_CHUNKS), mact[0, 0], act[0, 0])
